# Initial kernel scaffold; baseline (speedup 1.0000x reference)
import jax
import jax.numpy as jnp
from jax.experimental import pallas as pl


def kernel(x, edge_index, batch, W1, b1, W2, b2, Wfc, bfc):
    raise NotImplementedError("write your pallas kernel here")


# speedup vs baseline: 8.6180x; 8.6180x over previous
import functools

import jax
import jax.numpy as jnp
from jax import lax
from jax.experimental import pallas as pl
from jax.experimental.pallas import tpu as pltpu
from jax.experimental.pallas import tpu_sc as plsc

N = 10000
E = 160000
F_IN = 256
H1 = 64
H2 = 128
C = 10
B = 64

NC = 2
NS = 16
NW = NC * NS
K = 128
EPW = 5120
CHUNKS = EPW // K
E_PAD = NW * EPW
N_PAD = 10240
RPT = N_PAD // NS
R = 1024
GRID = N_PAD // R


def _make_sc_scatter(H):
  mesh = plsc.VectorSubcoreMesh(core_axis_name="c", subcore_axis_name="s")

  @functools.partial(
      pl.kernel,
      out_type=jax.ShapeDtypeStruct((NC, N_PAD, H), jnp.float32),
      mesh=mesh,
      scratch_types=[
          pltpu.VMEM((CHUNKS, K), jnp.int32),
          pltpu.VMEM((CHUNKS, K), jnp.int32),
          pltpu.VMEM((K, H), jnp.float32),
          pltpu.VMEM_SHARED((N_PAD, H), jnp.float32),
      ],
      compiler_params=pltpu.CompilerParams(use_tc_tiling_on_sc=False),
  )
  def scat(table_hbm, src_hbm, dst_hbm, zeros_hbm, out_hbm,
           src_v, dst_v, rows_v, acc_sh):
    c = lax.axis_index("c")
    s = lax.axis_index("s")
    wid = s * NC + c
    pltpu.sync_copy(src_hbm.at[wid], src_v)
    pltpu.sync_copy(dst_hbm.at[wid], dst_v)
    pltpu.sync_copy(zeros_hbm.at[pl.ds(s * RPT, RPT)],
                    acc_sh.at[pl.ds(s * RPT, RPT)])
    plsc.subcore_barrier()

    def body(j, carry):
      pltpu.sync_copy(table_hbm.at[src_v.at[j]], rows_v)
      pltpu.sync_copy(rows_v, acc_sh.at[dst_v.at[j]], add=True)
      return carry

    lax.fori_loop(0, CHUNKS, body, 0)
    plsc.subcore_barrier()
    pltpu.sync_copy(acc_sh.at[pl.ds(s * RPT, RPT)],
                    out_hbm.at[c, pl.ds(s * RPT, RPT)])

  return scat


_sc_scatter_h1 = _make_sc_scatter(H1)
_sc_scatter_h2 = _make_sc_scatter(H2)

DEGW = 16


def _make_sc_degree():
  mesh = plsc.VectorSubcoreMesh(core_axis_name="c", subcore_axis_name="s")

  @functools.partial(
      pl.kernel,
      out_type=jax.ShapeDtypeStruct((NC, N_PAD, DEGW), jnp.float32),
      mesh=mesh,
      scratch_types=[
          pltpu.VMEM((CHUNKS, K), jnp.int32),
          pltpu.VMEM((K, DEGW), jnp.float32),
          pltpu.VMEM_SHARED((N_PAD, DEGW), jnp.float32),
      ],
      compiler_params=pltpu.CompilerParams(use_tc_tiling_on_sc=False),
  )
  def deg(dst_hbm, ones_hbm, zeros_hbm, out_hbm, dst_v, ones_v, acc_sh):
    c = lax.axis_index("c")
    s = lax.axis_index("s")
    wid = s * NC + c
    pltpu.sync_copy(dst_hbm.at[wid], dst_v)
    pltpu.sync_copy(ones_hbm, ones_v)
    pltpu.sync_copy(zeros_hbm.at[pl.ds(s * RPT, RPT)],
                    acc_sh.at[pl.ds(s * RPT, RPT)])
    plsc.subcore_barrier()

    def body(j, carry):
      pltpu.sync_copy(ones_v, acc_sh.at[dst_v.at[j]], add=True)
      return carry

    lax.fori_loop(0, CHUNKS, body, 0)
    plsc.subcore_barrier()
    pltpu.sync_copy(acc_sh.at[pl.ds(s * RPT, RPT)],
                    out_hbm.at[c, pl.ds(s * RPT, RPT)])

  return deg


_sc_degree = _make_sc_degree()


def _tc1_body(x_ref, w1_ref, d0_ref, d1_ref, y1_ref):
  dinv = lax.rsqrt(1.0 + d0_ref[...] + d1_ref[...])
  y1_ref[...] = dinv * jnp.dot(x_ref[...], w1_ref[...],
                               preferred_element_type=jnp.float32)


def _tc1(x_p, W1, d0, d1):
  return pl.pallas_call(
      _tc1_body,
      grid=(GRID,),
      in_specs=[
          pl.BlockSpec((R, F_IN), lambda i: (i, 0)),
          pl.BlockSpec((F_IN, H1), lambda i: (0, 0)),
          pl.BlockSpec((R, 1), lambda i: (i, 0)),
          pl.BlockSpec((R, 1), lambda i: (i, 0)),
      ],
      out_specs=pl.BlockSpec((R, H1), lambda i: (i, 0)),
      out_shape=jax.ShapeDtypeStruct((N_PAD, H1), jnp.float32),
  )(x_p, W1, d0, d1)


def _tc2_body(p0_ref, p1_ref, y1_ref, d0_ref, d1_ref, w2_ref, b1_ref, y2_ref):
  dinv = lax.rsqrt(1.0 + d0_ref[...] + d1_ref[...])
  h1 = jnp.maximum(dinv * (p0_ref[...] + p1_ref[...] + y1_ref[...])
                   + b1_ref[...], 0.0)
  y2_ref[...] = dinv * jnp.dot(h1, w2_ref[...],
                               preferred_element_type=jnp.float32)


def _tc2(p0, p1, y1, d0, d1, W2, b1_2d):
  return pl.pallas_call(
      _tc2_body,
      grid=(GRID,),
      in_specs=[
          pl.BlockSpec((R, H1), lambda i: (i, 0)),
          pl.BlockSpec((R, H1), lambda i: (i, 0)),
          pl.BlockSpec((R, H1), lambda i: (i, 0)),
          pl.BlockSpec((R, 1), lambda i: (i, 0)),
          pl.BlockSpec((R, 1), lambda i: (i, 0)),
          pl.BlockSpec((H1, H2), lambda i: (0, 0)),
          pl.BlockSpec((1, H1), lambda i: (0, 0)),
      ],
      out_specs=pl.BlockSpec((R, H2), lambda i: (i, 0)),
      out_shape=jax.ShapeDtypeStruct((N_PAD, H2), jnp.float32),
  )(p0, p1, y1, d0, d1, W2, b1_2d)


def _tc3_body(q0_ref, q1_ref, y2_ref, d0_ref, d1_ref, b2_ref, batch_ref,
              wfc_ref, bfc_ref, out_ref, acc_ref, cnt_ref):
  i = pl.program_id(0)

  @pl.when(i == 0)
  def _init():
    acc_ref[...] = jnp.zeros_like(acc_ref)
    cnt_ref[...] = jnp.zeros_like(cnt_ref)

  dinv = lax.rsqrt(1.0 + d0_ref[...] + d1_ref[...])
  h2 = jnp.maximum(dinv * (q0_ref[...] + q1_ref[...] + y2_ref[...])
                   + b2_ref[...], 0.0)
  oh = (batch_ref[...] == lax.broadcasted_iota(jnp.int32, (R, B), 1)
        ).astype(jnp.float32)
  dn = (((0,), (0,)), ((), ()))
  acc_ref[...] += lax.dot_general(oh, h2, dn,
                                  preferred_element_type=jnp.float32)
  cnt_ref[...] += lax.dot_general(oh, jnp.ones((R, 1), jnp.float32), dn,
                                  preferred_element_type=jnp.float32)

  @pl.when(i == GRID - 1)
  def _finish():
    pooled = acc_ref[...] / jnp.maximum(cnt_ref[...], 1.0)
    logits = jnp.dot(pooled, wfc_ref[...],
                     preferred_element_type=jnp.float32) + bfc_ref[...]
    m = jnp.max(logits, axis=1, keepdims=True)
    sh = logits - m
    lse = jnp.log(jnp.sum(jnp.exp(sh), axis=1, keepdims=True))
    out_ref[...] = sh - lse


def _tc3(q0, q1, y2, d0, d1, b2_2d, batch_p, Wfc, bfc_2d):
  return pl.pallas_call(
      _tc3_body,
      grid=(GRID,),
      in_specs=[
          pl.BlockSpec((R, H2), lambda i: (i, 0)),
          pl.BlockSpec((R, H2), lambda i: (i, 0)),
          pl.BlockSpec((R, H2), lambda i: (i, 0)),
          pl.BlockSpec((R, 1), lambda i: (i, 0)),
          pl.BlockSpec((R, 1), lambda i: (i, 0)),
          pl.BlockSpec((1, H2), lambda i: (0, 0)),
          pl.BlockSpec((R, 1), lambda i: (i, 0)),
          pl.BlockSpec((H2, C), lambda i: (0, 0)),
          pl.BlockSpec((1, C), lambda i: (0, 0)),
      ],
      out_specs=pl.BlockSpec((B, C), lambda i: (0, 0)),
      out_shape=jax.ShapeDtypeStruct((B, C), jnp.float32),
      scratch_shapes=[
          pltpu.VMEM((B, H2), jnp.float32),
          pltpu.VMEM((B, 1), jnp.float32),
      ],
  )(q0, q1, y2, d0, d1, b2_2d, batch_p, Wfc, bfc_2d)


def kernel(x, edge_index, batch, W1, b1, W2, b2, Wfc, bfc):
  f32, i32 = jnp.float32, jnp.int32
  src = edge_index[0].astype(i32)
  dst = edge_index[1].astype(i32)
  pad_e = E_PAD - E
  src_p = jnp.concatenate([src, jnp.full((pad_e,), N, i32)]).reshape(
      NW, CHUNKS, K)
  dst_p = jnp.concatenate([dst, jnp.full((pad_e,), N, i32)]).reshape(
      NW, CHUNKS, K)
  x_p = jnp.concatenate([x.astype(f32),
                         jnp.zeros((N_PAD - N, F_IN), f32)], axis=0)
  batch_p = jnp.concatenate([batch.astype(i32),
                             jnp.full((N_PAD - N,), B, i32)]).reshape(
      N_PAD, 1)

  degp = _sc_degree(dst_p, jnp.ones((K, DEGW), f32),
                    jnp.zeros((N_PAD, DEGW), f32))
  d0, d1 = degp[0, :, :1], degp[1, :, :1]

  y1 = _tc1(x_p, W1.astype(f32), d0, d1)
  p = _sc_scatter_h1(y1, src_p, dst_p, jnp.zeros((N_PAD, H1), f32))
  y2 = _tc2(p[0], p[1], y1, d0, d1, W2.astype(f32), b1.reshape(1, H1))
  q = _sc_scatter_h2(y2, src_p, dst_p, jnp.zeros((N_PAD, H2), f32))
  return _tc3(q[0], q[1], y2, d0, d1, b2.reshape(1, H2), batch_p,
              Wfc.astype(f32), bfc.reshape(1, C))


# speedup vs baseline: 13.0266x; 1.5116x over previous
import functools

import jax
import jax.numpy as jnp
from jax import lax
from jax.experimental import pallas as pl
from jax.experimental.pallas import tpu as pltpu
from jax.experimental.pallas import tpu_sc as plsc

N = 10000
E = 160000
F_IN = 256
H1 = 64
H2 = 128
C = 10
B = 64

NC = 2
NS = 16
NW = NC * NS
K = 128
EPW = 5120
CHUNKS = EPW // K
E_PAD = NW * EPW
N_PAD = 10240
RPT = N_PAD // NS
R = 1024
GRID = N_PAD // R


NB = 4
CPT = E_PAD // NS // K


def _make_sc_scatter(H):
  Hh = H // NC
  mesh = plsc.VectorSubcoreMesh(core_axis_name="c", subcore_axis_name="s")

  @functools.partial(
      pl.kernel,
      out_type=jax.ShapeDtypeStruct((NC, N_PAD, Hh), jnp.float32),
      mesh=mesh,
      scratch_types=[
          pltpu.VMEM((CPT, K), jnp.int32),
          pltpu.VMEM((CPT, K), jnp.int32),
          pltpu.VMEM((NB, K, Hh), jnp.float32),
          pltpu.VMEM_SHARED((N_PAD, Hh), jnp.float32),
          pltpu.SemaphoreType.DMA,
          pltpu.SemaphoreType.DMA,
      ],
      compiler_params=pltpu.CompilerParams(use_tc_tiling_on_sc=False),
  )
  def scat(table_hbm, src_hbm, dst_hbm, zeros_hbm, out_hbm,
           src_v, dst_v, rows_v, acc_sh, gsem, ssem):
    c = lax.axis_index("c")
    s = lax.axis_index("s")
    pltpu.sync_copy(src_hbm.at[s], src_v)
    pltpu.sync_copy(dst_hbm.at[s], dst_v)
    pltpu.sync_copy(zeros_hbm.at[pl.ds(s * RPT, RPT)],
                    acc_sh.at[pl.ds(s * RPT, RPT)])
    plsc.subcore_barrier()

    def g_start(j, b):
      pltpu.async_copy(table_hbm.at[c].at[src_v.at[j]], rows_v.at[b], gsem)

    def g_wait(b):
      pltpu.make_async_copy(table_hbm.at[c].at[src_v.at[0]], rows_v.at[b],
                            gsem).wait()

    def s_start(j, b):
      pltpu.async_copy(rows_v.at[b], acc_sh.at[dst_v.at[j]], ssem, add=True)

    def s_wait(b):
      pltpu.make_async_copy(rows_v.at[b], acc_sh.at[dst_v.at[0]],
                            ssem).wait()

    for b in range(NB):
      g_start(b, b)

    @pl.loop(0, CPT - NB, step=NB)
    def _group(i):
      for b in range(NB):
        g_wait(b)
        s_start(i + b, b)
      for b in range(NB):
        s_wait(b)
        g_start(i + NB + b, b)

    for b in range(NB):
      g_wait(b)
      s_start(CPT - NB + b, b)
    for b in range(NB):
      s_wait(b)

    plsc.subcore_barrier()
    pltpu.sync_copy(acc_sh.at[pl.ds(s * RPT, RPT)],
                    out_hbm.at[c, pl.ds(s * RPT, RPT)])

  return scat


_sc_scatter_h1 = _make_sc_scatter(H1)
_sc_scatter_h2 = _make_sc_scatter(H2)

DEGW = 16


def _make_sc_degree():
  mesh = plsc.VectorSubcoreMesh(core_axis_name="c", subcore_axis_name="s")

  @functools.partial(
      pl.kernel,
      out_type=jax.ShapeDtypeStruct((NC, N_PAD, DEGW), jnp.float32),
      mesh=mesh,
      scratch_types=[
          pltpu.VMEM((CHUNKS, K), jnp.int32),
          pltpu.VMEM((K, DEGW), jnp.float32),
          pltpu.VMEM_SHARED((N_PAD, DEGW), jnp.float32),
      ],
      compiler_params=pltpu.CompilerParams(use_tc_tiling_on_sc=False),
  )
  def deg(dst_hbm, ones_hbm, zeros_hbm, out_hbm, dst_v, ones_v, acc_sh):
    c = lax.axis_index("c")
    s = lax.axis_index("s")
    wid = s * NC + c
    pltpu.sync_copy(dst_hbm.at[wid], dst_v)
    pltpu.sync_copy(ones_hbm, ones_v)
    pltpu.sync_copy(zeros_hbm.at[pl.ds(s * RPT, RPT)],
                    acc_sh.at[pl.ds(s * RPT, RPT)])
    plsc.subcore_barrier()

    def body(j, carry):
      pltpu.sync_copy(ones_v, acc_sh.at[dst_v.at[j]], add=True)
      return carry

    lax.fori_loop(0, CHUNKS, body, 0)
    plsc.subcore_barrier()
    pltpu.sync_copy(acc_sh.at[pl.ds(s * RPT, RPT)],
                    out_hbm.at[c, pl.ds(s * RPT, RPT)])

  return deg


_sc_degree = _make_sc_degree()


HH1 = H1 // NC
HH2 = H2 // NC


def _split(y, hh, out_ref):
  out_ref[0] = y[:, :hh]
  out_ref[1] = y[:, hh:]


def _tc1_body(x_ref, w1_ref, d0_ref, d1_ref, y1_ref):
  dinv = lax.rsqrt(1.0 + d0_ref[...] + d1_ref[...])
  y1 = dinv * jnp.dot(x_ref[...], w1_ref[...],
                      preferred_element_type=jnp.float32)
  _split(y1, HH1, y1_ref)


def _tc1(x_p, W1, d0, d1):
  return pl.pallas_call(
      _tc1_body,
      grid=(GRID,),
      in_specs=[
          pl.BlockSpec((R, F_IN), lambda i: (i, 0)),
          pl.BlockSpec((F_IN, H1), lambda i: (0, 0)),
          pl.BlockSpec((R, 1), lambda i: (i, 0)),
          pl.BlockSpec((R, 1), lambda i: (i, 0)),
      ],
      out_specs=pl.BlockSpec((NC, R, HH1), lambda i: (0, i, 0)),
      out_shape=jax.ShapeDtypeStruct((NC, N_PAD, HH1), jnp.float32),
  )(x_p, W1, d0, d1)


def _tc2_body(p_ref, y1_ref, d0_ref, d1_ref, w2_ref, b1_ref, y2_ref):
  dinv = lax.rsqrt(1.0 + d0_ref[...] + d1_ref[...])
  t = p_ref[...] + y1_ref[...]
  h1 = jnp.maximum(
      dinv * jnp.concatenate([t[0], t[1]], axis=1) + b1_ref[...], 0.0)
  y2 = dinv * jnp.dot(h1, w2_ref[...], preferred_element_type=jnp.float32)
  _split(y2, HH2, y2_ref)


def _tc2(p, y1, d0, d1, W2, b1_2d):
  return pl.pallas_call(
      _tc2_body,
      grid=(GRID,),
      in_specs=[
          pl.BlockSpec((NC, R, HH1), lambda i: (0, i, 0)),
          pl.BlockSpec((NC, R, HH1), lambda i: (0, i, 0)),
          pl.BlockSpec((R, 1), lambda i: (i, 0)),
          pl.BlockSpec((R, 1), lambda i: (i, 0)),
          pl.BlockSpec((H1, H2), lambda i: (0, 0)),
          pl.BlockSpec((1, H1), lambda i: (0, 0)),
      ],
      out_specs=pl.BlockSpec((NC, R, HH2), lambda i: (0, i, 0)),
      out_shape=jax.ShapeDtypeStruct((NC, N_PAD, HH2), jnp.float32),
  )(p, y1, d0, d1, W2, b1_2d)


def _tc3_body(q_ref, y2_ref, d0_ref, d1_ref, b2_ref, batch_ref,
              wfc_ref, bfc_ref, out_ref, acc_ref, cnt_ref):
  i = pl.program_id(0)

  @pl.when(i == 0)
  def _init():
    acc_ref[...] = jnp.zeros_like(acc_ref)
    cnt_ref[...] = jnp.zeros_like(cnt_ref)

  dinv = lax.rsqrt(1.0 + d0_ref[...] + d1_ref[...])
  t = q_ref[...] + y2_ref[...]
  h2 = jnp.maximum(
      dinv * jnp.concatenate([t[0], t[1]], axis=1) + b2_ref[...], 0.0)
  oh = (batch_ref[...] == lax.broadcasted_iota(jnp.int32, (R, B), 1)
        ).astype(jnp.float32)
  dn = (((0,), (0,)), ((), ()))
  acc_ref[...] += lax.dot_general(oh, h2, dn,
                                  preferred_element_type=jnp.float32)
  cnt_ref[...] += lax.dot_general(oh, jnp.ones((R, 1), jnp.float32), dn,
                                  preferred_element_type=jnp.float32)

  @pl.when(i == GRID - 1)
  def _finish():
    pooled = acc_ref[...] / jnp.maximum(cnt_ref[...], 1.0)
    logits = jnp.dot(pooled, wfc_ref[...],
                     preferred_element_type=jnp.float32) + bfc_ref[...]
    m = jnp.max(logits, axis=1, keepdims=True)
    sh = logits - m
    lse = jnp.log(jnp.sum(jnp.exp(sh), axis=1, keepdims=True))
    out_ref[...] = sh - lse


def _tc3(q, y2, d0, d1, b2_2d, batch_p, Wfc, bfc_2d):
  return pl.pallas_call(
      _tc3_body,
      grid=(GRID,),
      in_specs=[
          pl.BlockSpec((NC, R, HH2), lambda i: (0, i, 0)),
          pl.BlockSpec((NC, R, HH2), lambda i: (0, i, 0)),
          pl.BlockSpec((R, 1), lambda i: (i, 0)),
          pl.BlockSpec((R, 1), lambda i: (i, 0)),
          pl.BlockSpec((1, H2), lambda i: (0, 0)),
          pl.BlockSpec((R, 1), lambda i: (i, 0)),
          pl.BlockSpec((H2, C), lambda i: (0, 0)),
          pl.BlockSpec((1, C), lambda i: (0, 0)),
      ],
      out_specs=pl.BlockSpec((B, C), lambda i: (0, 0)),
      out_shape=jax.ShapeDtypeStruct((B, C), jnp.float32),
      scratch_shapes=[
          pltpu.VMEM((B, H2), jnp.float32),
          pltpu.VMEM((B, 1), jnp.float32),
      ],
  )(q, y2, d0, d1, b2_2d, batch_p, Wfc, bfc_2d)


def kernel(x, edge_index, batch, W1, b1, W2, b2, Wfc, bfc):
  f32, i32 = jnp.float32, jnp.int32
  src = edge_index[0].astype(i32)
  dst = edge_index[1].astype(i32)
  pad_e = E_PAD - E
  src_p = jnp.concatenate([src, jnp.full((pad_e,), N, i32)]).reshape(
      NS, CPT, K)
  dst_p = jnp.concatenate([dst, jnp.full((pad_e,), N, i32)]).reshape(
      NS, CPT, K)
  dst_w = dst_p.reshape(NW, CHUNKS, K)
  x_p = jnp.concatenate([x.astype(f32),
                         jnp.zeros((N_PAD - N, F_IN), f32)], axis=0)
  batch_p = jnp.concatenate([batch.astype(i32),
                             jnp.full((N_PAD - N,), B, i32)]).reshape(
      N_PAD, 1)

  degp = _sc_degree(dst_w, jnp.ones((K, DEGW), f32),
                    jnp.zeros((N_PAD, DEGW), f32))
  d0, d1 = degp[0, :, :1], degp[1, :, :1]

  y1 = _tc1(x_p, W1.astype(f32), d0, d1)
  p = _sc_scatter_h1(y1, src_p, dst_p, jnp.zeros((N_PAD, HH1), f32))
  y2 = _tc2(p, y1, d0, d1, W2.astype(f32), b1.reshape(1, H1))
  q = _sc_scatter_h2(y2, src_p, dst_p, jnp.zeros((N_PAD, HH2), f32))
  return _tc3(q, y2, d0, d1, b2.reshape(1, H2), batch_p,
              Wfc.astype(f32), bfc.reshape(1, C))


# speedup vs baseline: 13.0909x; 1.0049x over previous
import functools

import jax
import jax.numpy as jnp
from jax import lax
from jax.experimental import pallas as pl
from jax.experimental.pallas import tpu as pltpu
from jax.experimental.pallas import tpu_sc as plsc

N = 10000
E = 160000
F_IN = 256
H1 = 64
H2 = 128
C = 10
B = 64

NC = 2
NS = 16
NW = NC * NS
K = 128
EPW = 5120
CHUNKS = EPW // K
E_PAD = NW * EPW
N_PAD = 10240
RPT = N_PAD // NS
R = 1024
GRID = N_PAD // R


CPT = E_PAD // NS // K


def _make_sc_scatter(H, NB):
  Hh = H // NC
  mesh = plsc.VectorSubcoreMesh(core_axis_name="c", subcore_axis_name="s")

  @functools.partial(
      pl.kernel,
      out_type=jax.ShapeDtypeStruct((NC, N_PAD, Hh), jnp.float32),
      mesh=mesh,
      scratch_types=[
          pltpu.VMEM((CPT, K), jnp.int32),
          pltpu.VMEM((CPT, K), jnp.int32),
          pltpu.VMEM((NB, K, Hh), jnp.float32),
          pltpu.VMEM_SHARED((N_PAD, Hh), jnp.float32),
          pltpu.SemaphoreType.DMA,
          pltpu.SemaphoreType.DMA,
      ],
      compiler_params=pltpu.CompilerParams(use_tc_tiling_on_sc=False),
  )
  def scat(table_hbm, src_hbm, dst_hbm, zeros_hbm, out_hbm,
           src_v, dst_v, rows_v, acc_sh, gsem, ssem):
    c = lax.axis_index("c")
    s = lax.axis_index("s")
    pltpu.sync_copy(src_hbm.at[s], src_v)
    pltpu.sync_copy(dst_hbm.at[s], dst_v)
    pltpu.sync_copy(zeros_hbm.at[pl.ds(s * RPT, RPT)],
                    acc_sh.at[pl.ds(s * RPT, RPT)])
    plsc.subcore_barrier()

    def g_start(j, b):
      pltpu.async_copy(table_hbm.at[c].at[src_v.at[j]], rows_v.at[b], gsem)

    def g_wait(b):
      pltpu.make_async_copy(table_hbm.at[c].at[src_v.at[0]], rows_v.at[b],
                            gsem).wait()

    def s_start(j, b):
      pltpu.async_copy(rows_v.at[b], acc_sh.at[dst_v.at[j]], ssem, add=True)

    def s_wait(b):
      pltpu.make_async_copy(rows_v.at[b], acc_sh.at[dst_v.at[0]],
                            ssem).wait()

    for b in range(NB):
      g_start(b, b)

    @pl.loop(0, CPT - NB, step=NB)
    def _group(i):
      for b in range(NB):
        g_wait(b)
        s_start(i + b, b)
      for b in range(NB):
        s_wait(b)
        g_start(i + NB + b, b)

    for b in range(NB):
      g_wait(b)
      s_start(CPT - NB + b, b)
    for b in range(NB):
      s_wait(b)

    plsc.subcore_barrier()
    pltpu.sync_copy(acc_sh.at[pl.ds(s * RPT, RPT)],
                    out_hbm.at[c, pl.ds(s * RPT, RPT)])

  return scat


_sc_scatter_h1 = _make_sc_scatter(H1, 8)
_sc_scatter_h2 = _make_sc_scatter(H2, 5)

DEGW = 16


def _make_sc_degree():
  mesh = plsc.VectorSubcoreMesh(core_axis_name="c", subcore_axis_name="s")

  @functools.partial(
      pl.kernel,
      out_type=jax.ShapeDtypeStruct((NC, N_PAD, DEGW), jnp.float32),
      mesh=mesh,
      scratch_types=[
          pltpu.VMEM((CHUNKS, K), jnp.int32),
          pltpu.VMEM((K, DEGW), jnp.float32),
          pltpu.VMEM_SHARED((N_PAD, DEGW), jnp.float32),
      ],
      compiler_params=pltpu.CompilerParams(use_tc_tiling_on_sc=False),
  )
  def deg(dst_hbm, ones_hbm, zeros_hbm, out_hbm, dst_v, ones_v, acc_sh):
    c = lax.axis_index("c")
    s = lax.axis_index("s")
    wid = s * NC + c
    pltpu.sync_copy(dst_hbm.at[wid], dst_v)
    pltpu.sync_copy(ones_hbm, ones_v)
    pltpu.sync_copy(zeros_hbm.at[pl.ds(s * RPT, RPT)],
                    acc_sh.at[pl.ds(s * RPT, RPT)])
    plsc.subcore_barrier()

    def body(j, carry):
      pltpu.sync_copy(ones_v, acc_sh.at[dst_v.at[j]], add=True)
      return carry

    lax.fori_loop(0, CHUNKS, body, 0)
    plsc.subcore_barrier()
    pltpu.sync_copy(acc_sh.at[pl.ds(s * RPT, RPT)],
                    out_hbm.at[c, pl.ds(s * RPT, RPT)])

  return deg


_sc_degree = _make_sc_degree()


HH1 = H1 // NC
HH2 = H2 // NC


def _split(y, hh, out_ref):
  out_ref[0] = y[:, :hh]
  out_ref[1] = y[:, hh:]


def _tc1_body(x_ref, w1_ref, d0_ref, d1_ref, y1_ref):
  dinv = lax.rsqrt(1.0 + d0_ref[...] + d1_ref[...])
  y1 = dinv * jnp.dot(x_ref[...], w1_ref[...],
                      preferred_element_type=jnp.float32)
  _split(y1, HH1, y1_ref)


def _tc1(x_p, W1, d0, d1):
  return pl.pallas_call(
      _tc1_body,
      grid=(GRID,),
      in_specs=[
          pl.BlockSpec((R, F_IN), lambda i: (i, 0)),
          pl.BlockSpec((F_IN, H1), lambda i: (0, 0)),
          pl.BlockSpec((R, 1), lambda i: (i, 0)),
          pl.BlockSpec((R, 1), lambda i: (i, 0)),
      ],
      out_specs=pl.BlockSpec((NC, R, HH1), lambda i: (0, i, 0)),
      out_shape=jax.ShapeDtypeStruct((NC, N_PAD, HH1), jnp.float32),
  )(x_p, W1, d0, d1)


def _tc2_body(p_ref, y1_ref, d0_ref, d1_ref, w2_ref, b1_ref, y2_ref):
  dinv = lax.rsqrt(1.0 + d0_ref[...] + d1_ref[...])
  t = p_ref[...] + y1_ref[...]
  h1 = jnp.maximum(
      dinv * jnp.concatenate([t[0], t[1]], axis=1) + b1_ref[...], 0.0)
  y2 = dinv * jnp.dot(h1, w2_ref[...], preferred_element_type=jnp.float32)
  _split(y2, HH2, y2_ref)


def _tc2(p, y1, d0, d1, W2, b1_2d):
  return pl.pallas_call(
      _tc2_body,
      grid=(GRID,),
      in_specs=[
          pl.BlockSpec((NC, R, HH1), lambda i: (0, i, 0)),
          pl.BlockSpec((NC, R, HH1), lambda i: (0, i, 0)),
          pl.BlockSpec((R, 1), lambda i: (i, 0)),
          pl.BlockSpec((R, 1), lambda i: (i, 0)),
          pl.BlockSpec((H1, H2), lambda i: (0, 0)),
          pl.BlockSpec((1, H1), lambda i: (0, 0)),
      ],
      out_specs=pl.BlockSpec((NC, R, HH2), lambda i: (0, i, 0)),
      out_shape=jax.ShapeDtypeStruct((NC, N_PAD, HH2), jnp.float32),
  )(p, y1, d0, d1, W2, b1_2d)


def _tc3_body(q_ref, y2_ref, d0_ref, d1_ref, b2_ref, batch_ref,
              wfc_ref, bfc_ref, out_ref, acc_ref, cnt_ref):
  i = pl.program_id(0)

  @pl.when(i == 0)
  def _init():
    acc_ref[...] = jnp.zeros_like(acc_ref)
    cnt_ref[...] = jnp.zeros_like(cnt_ref)

  dinv = lax.rsqrt(1.0 + d0_ref[...] + d1_ref[...])
  t = q_ref[...] + y2_ref[...]
  h2 = jnp.maximum(
      dinv * jnp.concatenate([t[0], t[1]], axis=1) + b2_ref[...], 0.0)
  oh = (batch_ref[...] == lax.broadcasted_iota(jnp.int32, (R, B), 1)
        ).astype(jnp.float32)
  dn = (((0,), (0,)), ((), ()))
  acc_ref[...] += lax.dot_general(oh, h2, dn,
                                  preferred_element_type=jnp.float32)
  cnt_ref[...] += lax.dot_general(oh, jnp.ones((R, 1), jnp.float32), dn,
                                  preferred_element_type=jnp.float32)

  @pl.when(i == GRID - 1)
  def _finish():
    pooled = acc_ref[...] / jnp.maximum(cnt_ref[...], 1.0)
    logits = jnp.dot(pooled, wfc_ref[...],
                     preferred_element_type=jnp.float32) + bfc_ref[...]
    m = jnp.max(logits, axis=1, keepdims=True)
    sh = logits - m
    lse = jnp.log(jnp.sum(jnp.exp(sh), axis=1, keepdims=True))
    out_ref[...] = sh - lse


def _tc3(q, y2, d0, d1, b2_2d, batch_p, Wfc, bfc_2d):
  return pl.pallas_call(
      _tc3_body,
      grid=(GRID,),
      in_specs=[
          pl.BlockSpec((NC, R, HH2), lambda i: (0, i, 0)),
          pl.BlockSpec((NC, R, HH2), lambda i: (0, i, 0)),
          pl.BlockSpec((R, 1), lambda i: (i, 0)),
          pl.BlockSpec((R, 1), lambda i: (i, 0)),
          pl.BlockSpec((1, H2), lambda i: (0, 0)),
          pl.BlockSpec((R, 1), lambda i: (i, 0)),
          pl.BlockSpec((H2, C), lambda i: (0, 0)),
          pl.BlockSpec((1, C), lambda i: (0, 0)),
      ],
      out_specs=pl.BlockSpec((B, C), lambda i: (0, 0)),
      out_shape=jax.ShapeDtypeStruct((B, C), jnp.float32),
      scratch_shapes=[
          pltpu.VMEM((B, H2), jnp.float32),
          pltpu.VMEM((B, 1), jnp.float32),
      ],
  )(q, y2, d0, d1, b2_2d, batch_p, Wfc, bfc_2d)


def kernel(x, edge_index, batch, W1, b1, W2, b2, Wfc, bfc):
  f32, i32 = jnp.float32, jnp.int32
  src = edge_index[0].astype(i32)
  dst = edge_index[1].astype(i32)
  pad_e = E_PAD - E
  src_p = jnp.concatenate([src, jnp.full((pad_e,), N, i32)]).reshape(
      NS, CPT, K)
  dst_p = jnp.concatenate([dst, jnp.full((pad_e,), N, i32)]).reshape(
      NS, CPT, K)
  dst_w = dst_p.reshape(NW, CHUNKS, K)
  x_p = jnp.concatenate([x.astype(f32),
                         jnp.zeros((N_PAD - N, F_IN), f32)], axis=0)
  batch_p = jnp.concatenate([batch.astype(i32),
                             jnp.full((N_PAD - N,), B, i32)]).reshape(
      N_PAD, 1)

  degp = _sc_degree(dst_w, jnp.ones((K, DEGW), f32),
                    jnp.zeros((N_PAD, DEGW), f32))
  d0, d1 = degp[0, :, :1], degp[1, :, :1]

  y1 = _tc1(x_p, W1.astype(f32), d0, d1)
  p = _sc_scatter_h1(y1, src_p, dst_p, jnp.zeros((N_PAD, HH1), f32))
  y2 = _tc2(p, y1, d0, d1, W2.astype(f32), b1.reshape(1, H1))
  q = _sc_scatter_h2(y2, src_p, dst_p, jnp.zeros((N_PAD, HH2), f32))
  return _tc3(q, y2, d0, d1, b2.reshape(1, H2), batch_p,
              Wfc.astype(f32), bfc.reshape(1, C))


# speedup vs baseline: 13.2174x; 1.0097x over previous
import functools

import jax
import jax.numpy as jnp
from jax import lax
from jax.experimental import pallas as pl
from jax.experimental.pallas import tpu as pltpu
from jax.experimental.pallas import tpu_sc as plsc

N = 10000
E = 160000
F_IN = 256
H1 = 64
H2 = 128
C = 10
B = 64

NC = 2
NS = 16
NW = NC * NS
K = 128
EPW = 5120
CHUNKS = EPW // K
E_PAD = NW * EPW
N_PAD = 10240
RPT = N_PAD // NS
R = 1024
GRID = N_PAD // R


CPT = E_PAD // NS // K


def _make_sc_scatter(H, NB):
  Hh = H // NC
  mesh = plsc.VectorSubcoreMesh(core_axis_name="c", subcore_axis_name="s")

  @functools.partial(
      pl.kernel,
      out_type=jax.ShapeDtypeStruct((NC, N_PAD, Hh), jnp.float32),
      mesh=mesh,
      scratch_types=[
          pltpu.VMEM((CPT, K), jnp.int32),
          pltpu.VMEM((CPT, K), jnp.int32),
          pltpu.VMEM((NB, K, Hh), jnp.float32),
          pltpu.VMEM_SHARED((N_PAD, Hh), jnp.float32),
          pltpu.SemaphoreType.DMA,
          pltpu.SemaphoreType.DMA,
          pltpu.SemaphoreType.DMA,
      ],
      compiler_params=pltpu.CompilerParams(use_tc_tiling_on_sc=False),
  )
  def scat(table_hbm, src_hbm, dst_hbm, zeros_hbm, out_hbm,
           src_v, dst_v, rows_v, acc_sh, gsem, ssem, psem):
    c = lax.axis_index("c")
    s = lax.axis_index("s")
    pltpu.async_copy(zeros_hbm.at[pl.ds(s * RPT, RPT)],
                     acc_sh.at[pl.ds(s * RPT, RPT)], psem)
    isrc = pltpu.async_copy(src_hbm.at[s], src_v, gsem)
    idst = pltpu.async_copy(dst_hbm.at[s], dst_v, gsem)

    def g_start(j, b):
      pltpu.async_copy(table_hbm.at[c].at[src_v.at[j]], rows_v.at[b], gsem)

    def g_wait(b):
      pltpu.make_async_copy(table_hbm.at[c].at[src_v.at[0]], rows_v.at[b],
                            gsem).wait()

    def s_start(j, b):
      pltpu.async_copy(rows_v.at[b], acc_sh.at[dst_v.at[j]], ssem, add=True)

    def s_wait(b):
      pltpu.make_async_copy(rows_v.at[b], acc_sh.at[dst_v.at[0]],
                            ssem).wait()

    isrc.wait()
    idst.wait()
    for b in range(NB):
      g_start(b, b)
    pltpu.make_async_copy(
        zeros_hbm.at[pl.ds(s * RPT, RPT)],
        acc_sh.at[pl.ds(s * RPT, RPT)], psem).wait()
    plsc.subcore_barrier()

    @pl.loop(0, CPT - NB, step=NB)
    def _group(i):
      for b in range(NB):
        g_wait(b)
        s_start(i + b, b)
      for b in range(NB):
        s_wait(b)
        g_start(i + NB + b, b)

    for b in range(NB):
      g_wait(b)
      s_start(CPT - NB + b, b)
    for b in range(NB):
      s_wait(b)

    plsc.subcore_barrier()
    pltpu.sync_copy(acc_sh.at[pl.ds(s * RPT, RPT)],
                    out_hbm.at[c, pl.ds(s * RPT, RPT)])

  return scat


_sc_scatter_h1 = _make_sc_scatter(H1, 8)
_sc_scatter_h2 = _make_sc_scatter(H2, 5)

DEGW = 16


def _make_sc_degree():
  mesh = plsc.VectorSubcoreMesh(core_axis_name="c", subcore_axis_name="s")

  @functools.partial(
      pl.kernel,
      out_type=jax.ShapeDtypeStruct((NC, N_PAD, DEGW), jnp.float32),
      mesh=mesh,
      scratch_types=[
          pltpu.VMEM((CHUNKS, K), jnp.int32),
          pltpu.VMEM((K, DEGW), jnp.float32),
          pltpu.VMEM_SHARED((N_PAD, DEGW), jnp.float32),
      ],
      compiler_params=pltpu.CompilerParams(use_tc_tiling_on_sc=False),
  )
  def deg(dst_hbm, ones_hbm, zeros_hbm, out_hbm, dst_v, ones_v, acc_sh):
    c = lax.axis_index("c")
    s = lax.axis_index("s")
    wid = s * NC + c
    pltpu.sync_copy(dst_hbm.at[wid], dst_v)
    pltpu.sync_copy(ones_hbm, ones_v)
    pltpu.sync_copy(zeros_hbm.at[pl.ds(s * RPT, RPT)],
                    acc_sh.at[pl.ds(s * RPT, RPT)])
    plsc.subcore_barrier()

    def body(j, carry):
      pltpu.sync_copy(ones_v, acc_sh.at[dst_v.at[j]], add=True)
      return carry

    lax.fori_loop(0, CHUNKS, body, 0)
    plsc.subcore_barrier()
    pltpu.sync_copy(acc_sh.at[pl.ds(s * RPT, RPT)],
                    out_hbm.at[c, pl.ds(s * RPT, RPT)])

  return deg


_sc_degree = _make_sc_degree()


HH1 = H1 // NC
HH2 = H2 // NC


def _split(y, hh, out_ref):
  out_ref[0] = y[:, :hh]
  out_ref[1] = y[:, hh:]


def _tc1_body(x_ref, w1_ref, d0_ref, d1_ref, y1_ref):
  dinv = lax.rsqrt(1.0 + d0_ref[...] + d1_ref[...])
  y1 = dinv * jnp.dot(x_ref[...], w1_ref[...],
                      preferred_element_type=jnp.float32)
  _split(y1, HH1, y1_ref)


def _tc1(x_p, W1, d0, d1):
  return pl.pallas_call(
      _tc1_body,
      grid=(GRID,),
      in_specs=[
          pl.BlockSpec((R, F_IN), lambda i: (i, 0)),
          pl.BlockSpec((F_IN, H1), lambda i: (0, 0)),
          pl.BlockSpec((R, 1), lambda i: (i, 0)),
          pl.BlockSpec((R, 1), lambda i: (i, 0)),
      ],
      out_specs=pl.BlockSpec((NC, R, HH1), lambda i: (0, i, 0)),
      out_shape=jax.ShapeDtypeStruct((NC, N_PAD, HH1), jnp.float32),
  )(x_p, W1, d0, d1)


def _tc2_body(p_ref, y1_ref, d0_ref, d1_ref, w2_ref, b1_ref, y2_ref):
  dinv = lax.rsqrt(1.0 + d0_ref[...] + d1_ref[...])
  t = p_ref[...] + y1_ref[...]
  h1 = jnp.maximum(
      dinv * jnp.concatenate([t[0], t[1]], axis=1) + b1_ref[...], 0.0)
  y2 = dinv * jnp.dot(h1, w2_ref[...], preferred_element_type=jnp.float32)
  _split(y2, HH2, y2_ref)


def _tc2(p, y1, d0, d1, W2, b1_2d):
  return pl.pallas_call(
      _tc2_body,
      grid=(GRID,),
      in_specs=[
          pl.BlockSpec((NC, R, HH1), lambda i: (0, i, 0)),
          pl.BlockSpec((NC, R, HH1), lambda i: (0, i, 0)),
          pl.BlockSpec((R, 1), lambda i: (i, 0)),
          pl.BlockSpec((R, 1), lambda i: (i, 0)),
          pl.BlockSpec((H1, H2), lambda i: (0, 0)),
          pl.BlockSpec((1, H1), lambda i: (0, 0)),
      ],
      out_specs=pl.BlockSpec((NC, R, HH2), lambda i: (0, i, 0)),
      out_shape=jax.ShapeDtypeStruct((NC, N_PAD, HH2), jnp.float32),
  )(p, y1, d0, d1, W2, b1_2d)


def _tc3_body(q_ref, y2_ref, d0_ref, d1_ref, b2_ref, batch_ref,
              wfc_ref, bfc_ref, out_ref, acc_ref, cnt_ref):
  i = pl.program_id(0)

  @pl.when(i == 0)
  def _init():
    acc_ref[...] = jnp.zeros_like(acc_ref)
    cnt_ref[...] = jnp.zeros_like(cnt_ref)

  dinv = lax.rsqrt(1.0 + d0_ref[...] + d1_ref[...])
  t = q_ref[...] + y2_ref[...]
  h2 = jnp.maximum(
      dinv * jnp.concatenate([t[0], t[1]], axis=1) + b2_ref[...], 0.0)
  oh = (batch_ref[...] == lax.broadcasted_iota(jnp.int32, (R, B), 1)
        ).astype(jnp.float32)
  dn = (((0,), (0,)), ((), ()))
  acc_ref[...] += lax.dot_general(oh, h2, dn,
                                  preferred_element_type=jnp.float32)
  cnt_ref[...] += lax.dot_general(oh, jnp.ones((R, 1), jnp.float32), dn,
                                  preferred_element_type=jnp.float32)

  @pl.when(i == GRID - 1)
  def _finish():
    pooled = acc_ref[...] / jnp.maximum(cnt_ref[...], 1.0)
    logits = jnp.dot(pooled, wfc_ref[...],
                     preferred_element_type=jnp.float32) + bfc_ref[...]
    m = jnp.max(logits, axis=1, keepdims=True)
    sh = logits - m
    lse = jnp.log(jnp.sum(jnp.exp(sh), axis=1, keepdims=True))
    out_ref[...] = sh - lse


def _tc3(q, y2, d0, d1, b2_2d, batch_p, Wfc, bfc_2d):
  return pl.pallas_call(
      _tc3_body,
      grid=(GRID,),
      in_specs=[
          pl.BlockSpec((NC, R, HH2), lambda i: (0, i, 0)),
          pl.BlockSpec((NC, R, HH2), lambda i: (0, i, 0)),
          pl.BlockSpec((R, 1), lambda i: (i, 0)),
          pl.BlockSpec((R, 1), lambda i: (i, 0)),
          pl.BlockSpec((1, H2), lambda i: (0, 0)),
          pl.BlockSpec((R, 1), lambda i: (i, 0)),
          pl.BlockSpec((H2, C), lambda i: (0, 0)),
          pl.BlockSpec((1, C), lambda i: (0, 0)),
      ],
      out_specs=pl.BlockSpec((B, C), lambda i: (0, 0)),
      out_shape=jax.ShapeDtypeStruct((B, C), jnp.float32),
      scratch_shapes=[
          pltpu.VMEM((B, H2), jnp.float32),
          pltpu.VMEM((B, 1), jnp.float32),
      ],
  )(q, y2, d0, d1, b2_2d, batch_p, Wfc, bfc_2d)


def kernel(x, edge_index, batch, W1, b1, W2, b2, Wfc, bfc):
  f32, i32 = jnp.float32, jnp.int32
  src = edge_index[0].astype(i32)
  dst = edge_index[1].astype(i32)
  pad_e = E_PAD - E
  src_p = jnp.concatenate([src, jnp.full((pad_e,), N, i32)]).reshape(
      NS, CPT, K)
  dst_p = jnp.concatenate([dst, jnp.full((pad_e,), N, i32)]).reshape(
      NS, CPT, K)
  dst_w = dst_p.reshape(NW, CHUNKS, K)
  x_p = jnp.concatenate([x.astype(f32),
                         jnp.zeros((N_PAD - N, F_IN), f32)], axis=0)
  batch_p = jnp.concatenate([batch.astype(i32),
                             jnp.full((N_PAD - N,), B, i32)]).reshape(
      N_PAD, 1)

  degp = _sc_degree(dst_w, jnp.ones((K, DEGW), f32),
                    jnp.zeros((N_PAD, DEGW), f32))
  d0, d1 = degp[0, :, :1], degp[1, :, :1]

  y1 = _tc1(x_p, W1.astype(f32), d0, d1)
  p = _sc_scatter_h1(y1, src_p, dst_p, jnp.zeros((N_PAD, HH1), f32))
  y2 = _tc2(p, y1, d0, d1, W2.astype(f32), b1.reshape(1, H1))
  q = _sc_scatter_h2(y2, src_p, dst_p, jnp.zeros((N_PAD, HH2), f32))
  return _tc3(q, y2, d0, d1, b2.reshape(1, H2), batch_p,
              Wfc.astype(f32), bfc.reshape(1, C))
